# baseline (speedup 1.0000x reference)
import functools

import jax
import jax.numpy as jnp
from jax import lax
from jax.experimental import pallas as pl
from jax.experimental.pallas import tpu as pltpu
from jax.experimental.pallas import tpu_sc as plsc

N = 10000
E = 320000
D = 128
HD = D // 2
NP = 10240
NC = 2
NS = 16
LANE = 128
RT = 80
IB = 16
RPAD = NC * NS * RT
EPAD = RPAD * LANE
DUMP = N + 128
TROWS = NP // NS


@functools.cache
def _mesh():
    return plsc.VectorSubcoreMesh(core_axis_name="c", subcore_axis_name="s",
                                  num_cores=NC, num_subcores=NS)


def _pack_bf16(xp):
    u = jax.lax.bitcast_convert_type(xp.astype(jnp.bfloat16), jnp.uint16)
    w = u[:, :HD].astype(jnp.uint32) | (u[:, HD:].astype(jnp.uint32) << 16)
    return jax.lax.bitcast_convert_type(w, jnp.float32)


def _sc_agg(xpk, src_rows, dst_rows, zacc):
    @functools.partial(
        pl.kernel,
        out_type=jax.ShapeDtypeStruct((NC, NP, D), jnp.float32),
        mesh=_mesh(),
        compiler_params=pltpu.CompilerParams(use_tc_tiling_on_sc=False,
                                             needs_layout_passes=False),
        scratch_types=[
            pltpu.VMEM((IB, LANE), jnp.int32),
            pltpu.VMEM((IB, LANE), jnp.int32),
            pltpu.VMEM((LANE, HD), jnp.float32),
            pltpu.VMEM((LANE, HD), jnp.float32),
            pltpu.VMEM((LANE, D), jnp.float32),
            pltpu.VMEM_SHARED((NP, D), jnp.float32),
            pltpu.SemaphoreType.DMA,
            pltpu.SemaphoreType.DMA,
        ])
    def k(x_hbm, src_hbm, dst_hbm, za_hbm, sums_hbm,
          src_v, dst_v, g0, g1, u, acc_sh, sem0, sem1):
        c = lax.axis_index("c")
        s = lax.axis_index("s")
        r0 = s * TROWS
        pltpu.sync_copy(za_hbm.at[pl.ds(r0, TROWS)],
                        acc_sh.at[pl.ds(r0, TROWS)])
        base = (c * NS + s) * RT
        plsc.subcore_barrier()

        mask = jnp.full((16,), 0xFFFF0000, jnp.uint32).astype(jnp.int32)

        def start_gather(j, buf, sem):
            pltpu.async_copy(x_hbm.at[src_v.at[j]], buf, sem)

        def wait_gather(j, buf, sem):
            pltpu.make_async_copy(x_hbm.at[src_v.at[j]], buf, sem).wait()

        def unpack(buf):
            @pl.loop(0, LANE)
            def _(r):
                for kk in range(HD // 16):
                    v = plsc.bitcast(buf[r, pl.ds(16 * kk, 16)], jnp.int32)
                    u[r, pl.ds(16 * kk, 16)] = plsc.bitcast(
                        lax.shift_left(v, 16), jnp.float32)
                    u[r, pl.ds(HD + 16 * kk, 16)] = plsc.bitcast(
                        lax.bitwise_and(v, mask), jnp.float32)

        @pl.loop(0, RT // IB)
        def _(b):
            pltpu.sync_copy(src_hbm.at[pl.ds(base + b * IB, IB)], src_v)
            pltpu.sync_copy(dst_hbm.at[pl.ds(base + b * IB, IB)], dst_v)
            start_gather(0, g0, sem0)

            @pl.loop(0, IB - 1)
            def _(j):
                even = j % 2 == 0

                @pl.when(even)
                def _():
                    start_gather(j + 1, g1, sem1)
                    wait_gather(j, g0, sem0)
                    unpack(g0)
                    pltpu.sync_copy(u, acc_sh.at[dst_v.at[j]], add=True)

                @pl.when(jnp.logical_not(even))
                def _():
                    start_gather(j + 1, g0, sem0)
                    wait_gather(j, g1, sem1)
                    unpack(g1)
                    pltpu.sync_copy(u, acc_sh.at[dst_v.at[j]], add=True)

            last = IB - 1
            wait_gather(last, g1, sem1)
            unpack(g1)
            pltpu.sync_copy(u, acc_sh.at[dst_v.at[last]], add=True)

        plsc.subcore_barrier()
        pltpu.sync_copy(acc_sh.at[pl.ds(r0, TROWS)],
                        sums_hbm.at[c, pl.ds(r0, TROWS)])

    return k(xpk, src_rows, dst_rows, zacc)


def _sc_counts(dst_rows, zcnt, ones):
    @functools.partial(
        pl.kernel,
        out_type=jax.ShapeDtypeStruct((NC, NP, D), jnp.float32),
        mesh=_mesh(),
        scratch_types=[
            pltpu.VMEM((RT, LANE), jnp.int32),
            pltpu.VMEM((LANE, D), jnp.float32),
            pltpu.VMEM_SHARED((NP, D), jnp.float32),
        ])
    def k(dst_hbm, zc_hbm, ones_hbm, cnts_hbm, dst_v, ones_v, cnt_sh):
        c = lax.axis_index("c")
        s = lax.axis_index("s")
        r0 = s * TROWS
        pltpu.sync_copy(zc_hbm.at[pl.ds(r0, TROWS)],
                        cnt_sh.at[pl.ds(r0, TROWS)])
        pltpu.sync_copy(ones_hbm, ones_v)
        base = (c * NS + s) * RT
        pltpu.sync_copy(dst_hbm.at[pl.ds(base, RT)], dst_v)
        plsc.subcore_barrier()

        @pl.loop(0, RT)
        def _(j):
            pltpu.sync_copy(ones_v, cnt_sh.at[dst_v.at[j]], add=True)

        plsc.subcore_barrier()
        pltpu.sync_copy(cnt_sh.at[pl.ds(r0, TROWS)],
                        cnts_hbm.at[c, pl.ds(r0, TROWS)])

    return k(dst_rows, zcnt, ones)


_BM = 512


def _mm_body(x_ref, w_ref, o_ref):
    o_ref[...] = lax.dot_general(
        x_ref[...], w_ref[...], (((1,), (0,)), ((), ())),
        preferred_element_type=jnp.float32,
        precision=lax.Precision.HIGHEST)


def _tc_matmul(xp, wt):
    return pl.pallas_call(
        _mm_body,
        grid=(NP // _BM,),
        in_specs=[pl.BlockSpec((_BM, D), lambda i: (i, 0)),
                  pl.BlockSpec((D, D), lambda i: (0, 0))],
        out_specs=pl.BlockSpec((_BM, D), lambda i: (i, 0)),
        out_shape=jax.ShapeDtypeStruct((NP, D), jnp.float32),
    )(xp, wt)


def _combine_body(relu, s_ref, c_ref, xr_ref, wt_ref, b_ref, o_ref):
    cnt = jnp.maximum(c_ref[0, :, 0:1] + c_ref[1, :, 0:1], 1.0)
    mean = (s_ref[0] + s_ref[1]) / cnt
    y = lax.dot_general(mean, wt_ref[...], (((1,), (0,)), ((), ())),
                        preferred_element_type=jnp.float32,
                        precision=lax.Precision.HIGHEST)
    y = y + xr_ref[...] + b_ref[...]
    o_ref[...] = jnp.maximum(y, 0.0) if relu else y


def _tc_combine(sums, cnts, xr, wt, b, relu):
    return pl.pallas_call(
        functools.partial(_combine_body, relu),
        grid=(NP // _BM,),
        in_specs=[pl.BlockSpec((NC, _BM, D), lambda i: (0, i, 0)),
                  pl.BlockSpec((NC, _BM, D), lambda i: (0, i, 0)),
                  pl.BlockSpec((_BM, D), lambda i: (i, 0)),
                  pl.BlockSpec((D, D), lambda i: (0, 0)),
                  pl.BlockSpec((1, D), lambda i: (0, 0))],
        out_specs=pl.BlockSpec((_BM, D), lambda i: (i, 0)),
        out_shape=jax.ShapeDtypeStruct((NP, D), jnp.float32),
    )(sums, cnts, xr, wt, b)


def kernel(x, edge_index, Wl1, bl1, Wr1, Wl2, bl2, Wr2):
    src = edge_index[0].astype(jnp.int32)
    dst = edge_index[1].astype(jnp.int32)
    pad = EPAD - E
    src_rows = jnp.concatenate(
        [src, jnp.zeros((pad,), jnp.int32)]).reshape(RPAD, LANE)
    dst_rows = jnp.concatenate(
        [dst, jnp.full((pad,), DUMP, jnp.int32)]).reshape(RPAD, LANE)
    xp = jnp.pad(x, ((0, NP - N), (0, 0)))
    zacc = jnp.zeros((NP, D), jnp.float32)
    zcnt = jnp.zeros((NP, D), jnp.float32)

    c1 = _sc_counts(dst_rows, zcnt, jnp.ones((LANE, D), jnp.float32))
    s1 = _sc_agg(_pack_bf16(xp), src_rows, dst_rows, zacc)
    xr1 = _tc_matmul(xp, Wr1.T)
    h = _tc_combine(s1, c1, xr1, Wl1.T, bl1.reshape(1, D), relu=True)

    s2 = _sc_agg(_pack_bf16(h), src_rows, dst_rows, zacc)
    xr2 = _tc_matmul(h, Wr2.T)
    out = _tc_combine(s2, c1, xr2, Wl2.T, bl2.reshape(1, D), relu=False)
    return out[:N]


# speedup vs baseline: 1.0387x; 1.0387x over previous
import functools

import jax
import jax.numpy as jnp
from jax import lax
from jax.experimental import pallas as pl
from jax.experimental.pallas import tpu as pltpu
from jax.experimental.pallas import tpu_sc as plsc

N = 10000
E = 320000
D = 128
HD = D // 2
NP = 10240
NC = 2
NS = 16
LANE = 128
RT = 80
IB = 8
RPAD = NC * NS * RT
EPAD = RPAD * LANE
DUMP = N + 128
TROWS = NP // NS


@functools.cache
def _mesh():
    return plsc.VectorSubcoreMesh(core_axis_name="c", subcore_axis_name="s",
                                  num_cores=NC, num_subcores=NS)


def _pack_bf16(xp):
    u = jax.lax.bitcast_convert_type(xp.astype(jnp.bfloat16), jnp.uint16)
    w = u[:, :HD].astype(jnp.uint32) | (u[:, HD:].astype(jnp.uint32) << 16)
    return jax.lax.bitcast_convert_type(w, jnp.float32)


def _sc_agg(xpk, src_rows, dst_rows, zacc, zcnt=None, ones=None,
            with_counts=False):
    out_type = [jax.ShapeDtypeStruct((NC, NP, D), jnp.float32)]
    scratch = [
        pltpu.VMEM((IB, LANE), jnp.int32),
        pltpu.VMEM((IB, LANE), jnp.int32),
        pltpu.VMEM((LANE, HD), jnp.float32),
        pltpu.VMEM((LANE, HD), jnp.float32),
        pltpu.VMEM((LANE, D), jnp.float32),
        pltpu.VMEM_SHARED((NP, D), jnp.float32),
        pltpu.SemaphoreType.DMA,
        pltpu.SemaphoreType.DMA,
    ]
    if with_counts:
        out_type.append(jax.ShapeDtypeStruct((NC, NP, 16), jnp.float32))
        scratch += [
            pltpu.VMEM((LANE, 16), jnp.float32),
            pltpu.VMEM_SHARED((NP, 16), jnp.float32),
        ]

    @functools.partial(
        pl.kernel,
        out_type=tuple(out_type),
        mesh=_mesh(),
        compiler_params=pltpu.CompilerParams(use_tc_tiling_on_sc=False,
                                             needs_layout_passes=False),
        scratch_types=scratch)
    def k(x_hbm, src_hbm, dst_hbm, za_hbm, zc_hbm, ones_hbm, *refs):
        if with_counts:
            (sums_hbm, cnts_hbm, src_v, dst_v, g0, g1, u, acc_sh,
             sem0, sem1, ones_v, cnt_sh) = refs
        else:
            sums_hbm, src_v, dst_v, g0, g1, u, acc_sh, sem0, sem1 = refs
        c = lax.axis_index("c")
        s = lax.axis_index("s")
        r0 = s * TROWS
        pltpu.sync_copy(za_hbm.at[pl.ds(r0, TROWS)],
                        acc_sh.at[pl.ds(r0, TROWS)])
        if with_counts:
            pltpu.sync_copy(zc_hbm.at[pl.ds(r0, TROWS)],
                            cnt_sh.at[pl.ds(r0, TROWS)])
            pltpu.sync_copy(ones_hbm, ones_v)
        base = (c * NS + s) * RT
        plsc.subcore_barrier()

        mask = jnp.full((16,), 0xFFFF0000, jnp.uint32).astype(jnp.int32)

        def start_gather(j, buf, sem):
            pltpu.async_copy(x_hbm.at[src_v.at[j]], buf, sem)

        def wait_gather(j, buf, sem):
            pltpu.make_async_copy(x_hbm.at[src_v.at[j]], buf, sem).wait()

        def unpack(buf):
            @pl.loop(0, LANE)
            def _(r):
                for kk in range(HD // 16):
                    v = plsc.bitcast(buf[r, pl.ds(16 * kk, 16)], jnp.int32)
                    u[r, pl.ds(16 * kk, 16)] = plsc.bitcast(
                        lax.shift_left(v, 16), jnp.float32)
                    u[r, pl.ds(HD + 16 * kk, 16)] = plsc.bitcast(
                        lax.bitwise_and(v, mask), jnp.float32)

        @pl.loop(0, RT // IB)
        def _(b):
            pltpu.sync_copy(src_hbm.at[pl.ds(base + b * IB, IB)], src_v)
            pltpu.sync_copy(dst_hbm.at[pl.ds(base + b * IB, IB)], dst_v)
            start_gather(0, g0, sem0)

            @pl.loop(0, IB - 1)
            def _(j):
                even = j % 2 == 0

                @pl.when(even)
                def _():
                    start_gather(j + 1, g1, sem1)
                    wait_gather(j, g0, sem0)
                    unpack(g0)
                    pltpu.sync_copy(u, acc_sh.at[dst_v.at[j]], add=True)

                @pl.when(jnp.logical_not(even))
                def _():
                    start_gather(j + 1, g0, sem0)
                    wait_gather(j, g1, sem1)
                    unpack(g1)
                    pltpu.sync_copy(u, acc_sh.at[dst_v.at[j]], add=True)

                if with_counts:
                    pltpu.sync_copy(ones_v, cnt_sh.at[dst_v.at[j]], add=True)

            last = IB - 1
            wait_gather(last, g1, sem1)
            unpack(g1)
            pltpu.sync_copy(u, acc_sh.at[dst_v.at[last]], add=True)
            if with_counts:
                pltpu.sync_copy(ones_v, cnt_sh.at[dst_v.at[last]], add=True)

        plsc.subcore_barrier()
        pltpu.sync_copy(acc_sh.at[pl.ds(r0, TROWS)],
                        sums_hbm.at[c, pl.ds(r0, TROWS)])
        if with_counts:
            pltpu.sync_copy(cnt_sh.at[pl.ds(r0, TROWS)],
                            cnts_hbm.at[c, pl.ds(r0, TROWS)])

    res = k(xpk, src_rows, dst_rows, zacc,
            zcnt if zcnt is not None else jnp.zeros((8, 16), jnp.float32),
            ones if ones is not None else jnp.zeros((8, 16), jnp.float32))
    if with_counts:
        return res
    return res[0] if isinstance(res, (tuple, list)) else res


_BM = 512


def _mm_body(x_ref, w_ref, o_ref):
    o_ref[...] = lax.dot_general(
        x_ref[...], w_ref[...], (((1,), (0,)), ((), ())),
        preferred_element_type=jnp.float32,
        precision=lax.Precision.HIGHEST)


def _tc_matmul(xp, wt):
    return pl.pallas_call(
        _mm_body,
        grid=(NP // _BM,),
        in_specs=[pl.BlockSpec((_BM, D), lambda i: (i, 0)),
                  pl.BlockSpec((D, D), lambda i: (0, 0))],
        out_specs=pl.BlockSpec((_BM, D), lambda i: (i, 0)),
        out_shape=jax.ShapeDtypeStruct((NP, D), jnp.float32),
    )(xp, wt)


def _combine_body(relu, s_ref, c_ref, xr_ref, wt_ref, b_ref, o_ref):
    cnt = jnp.maximum(c_ref[0, :, 0:1] + c_ref[1, :, 0:1], 1.0)
    mean = (s_ref[0] + s_ref[1]) / cnt
    y = lax.dot_general(mean, wt_ref[...], (((1,), (0,)), ((), ())),
                        preferred_element_type=jnp.float32,
                        precision=lax.Precision.HIGHEST)
    y = y + xr_ref[...] + b_ref[...]
    o_ref[...] = jnp.maximum(y, 0.0) if relu else y


def _tc_combine(sums, cnts, xr, wt, b, relu):
    return pl.pallas_call(
        functools.partial(_combine_body, relu),
        grid=(NP // _BM,),
        in_specs=[pl.BlockSpec((NC, _BM, D), lambda i: (0, i, 0)),
                  pl.BlockSpec((NC, _BM, 16), lambda i: (0, i, 0)),
                  pl.BlockSpec((_BM, D), lambda i: (i, 0)),
                  pl.BlockSpec((D, D), lambda i: (0, 0)),
                  pl.BlockSpec((1, D), lambda i: (0, 0))],
        out_specs=pl.BlockSpec((_BM, D), lambda i: (i, 0)),
        out_shape=jax.ShapeDtypeStruct((NP, D), jnp.float32),
    )(sums, cnts, xr, wt, b)


def kernel(x, edge_index, Wl1, bl1, Wr1, Wl2, bl2, Wr2):
    src = edge_index[0].astype(jnp.int32)
    dst = edge_index[1].astype(jnp.int32)
    pad = EPAD - E
    src_rows = jnp.concatenate(
        [src, jnp.zeros((pad,), jnp.int32)]).reshape(RPAD, LANE)
    dst_rows = jnp.concatenate(
        [dst, jnp.full((pad,), DUMP, jnp.int32)]).reshape(RPAD, LANE)
    xp = jnp.pad(x, ((0, NP - N), (0, 0)))
    zacc = jnp.zeros((NP, D), jnp.float32)
    zcnt = jnp.zeros((NP, 16), jnp.float32)

    s1, c1 = _sc_agg(_pack_bf16(xp), src_rows, dst_rows, zacc, zcnt,
                     jnp.ones((LANE, 16), jnp.float32), with_counts=True)
    xr1 = _tc_matmul(xp, Wr1.T)
    h = _tc_combine(s1, c1, xr1, Wl1.T, bl1.reshape(1, D), relu=True)

    s2 = _sc_agg(_pack_bf16(h), src_rows, dst_rows, zacc)
    xr2 = _tc_matmul(h, Wr2.T)
    out = _tc_combine(s2, c1, xr2, Wl2.T, bl2.reshape(1, D), relu=False)
    return out[:N]
